# baseline (device time: 130484 ns/iter reference)
import jax
import jax.numpy as jnp
from jax import lax
from jax.experimental import pallas as pl
from jax.experimental.pallas import tpu as pltpu

_DeviceIdType = getattr(pl, "DeviceIdType", None) or pltpu.DeviceIdType
_sem_signal = getattr(pl, "semaphore_signal", None) or pltpu.semaphore_signal
_sem_wait = getattr(pl, "semaphore_wait", None) or pltpu.semaphore_wait

M = 2048
D = 2048
F = 8192
D_HALF = D // 2
F_HALF = F // 2
FC = 256
CHUNKS = [(i * FC, FC) for i in range(15)] + [(3840, 128), (3968, 128)]
NCH = len(CHUNKS)
XT = 512
NPB = 4


def kernel(x, dy):

    def body(x_ref, dy_ref, out_ref, x16, x_stage, dy_vmem, p_buf,
             red_stage, rs_recv,
             x_sems, dy_sems, out_sems,
             rs_send_sems, rs_recv_sems, ag_send_sems, ag_recv_sems):
        mx = lax.axis_index("x")
        my = lax.axis_index("y")
        x_nbr = (1 - mx, my)
        y_nbr = (mx, 1 - my)

        barrier = pltpu.get_barrier_semaphore()
        for nbr in (x_nbr, y_nbr):
            _sem_signal(barrier, inc=1, device_id=nbr,
                        device_id_type=_DeviceIdType.MESH)

        def dy_fetch(c):
            off, w = CHUNKS[c]
            cp = pltpu.make_async_copy(
                dy_ref.at[:, pl.ds(mx * F_HALF + off, w)],
                dy_vmem.at[c % 3, :, pl.ds(0, w)],
                dy_sems.at[c % 3],
            )
            cp.start()
            return cp

        dy_cps = [None] * NCH
        dy_cps[0] = dy_fetch(0)
        dy_cps[1] = dy_fetch(1)

        xcols = [
            (1 - my) * D_HALF, (1 - my) * D_HALF + XT,
            my * D_HALF, my * D_HALF + XT,
        ]

        def x_fetch(i):
            cp = pltpu.make_async_copy(
                x_ref.at[:, pl.ds(xcols[i], XT)], x_stage.at[i % 2],
                x_sems.at[i % 2],
            )
            cp.start()
            return cp

        def x_cast(i):
            x16[:, pl.ds(xcols[i], XT)] = x_stage[i % 2].astype(jnp.bfloat16)

        x_cp0 = x_fetch(0)
        x_cp1 = x_fetch(1)
        x_cp0.wait()
        x_cast(0)
        x_cp2 = x_fetch(2)
        x_cp1.wait()
        x_cast(1)

        rs_ops = [None] * NCH
        ag_ops = [None] * NCH
        out_cps = [None] * NCH

        def mm_half(c, b, row0):
            w = CHUNKS[c][1]
            rows = pl.ds(row0, D_HALF)
            p_buf[c % NPB, rows, pl.ds(0, w)] = lax.dot_general(
                x16[:, rows], b,
                dimension_numbers=(((0,), (0,)), ((), ())),
                preferred_element_type=jnp.float32,
            ).astype(jnp.bfloat16)

        def rs_start(c):
            w = CHUNKS[c][1]
            rs = pltpu.make_async_remote_copy(
                src_ref=p_buf.at[
                    c % NPB, pl.ds((1 - my) * D_HALF, D_HALF), pl.ds(0, w)
                ],
                dst_ref=rs_recv.at[c, :, pl.ds(0, w)],
                send_sem=rs_send_sems.at[c],
                recv_sem=rs_recv_sems.at[c],
                device_id=y_nbr,
                device_id_type=_DeviceIdType.MESH,
            )
            rs.start()
            rs_ops[c] = rs

        def finish(c):
            off, w = CHUNKS[c]
            slot = c % NPB
            if c >= NPB:
                out_cps[c - NPB].wait()
                ag_ops[c - NPB].wait_send()
            rs_ops[c].wait_recv()
            red_stage[slot, :, pl.ds(0, w)] = (
                p_buf[c % NPB, pl.ds(my * D_HALF, D_HALF), pl.ds(0, w)]
                + rs_recv[c, :, pl.ds(0, w)]
            )
            col = pl.ds(mx * F_HALF + off, w)
            cp = pltpu.make_async_copy(
                red_stage.at[slot, :, pl.ds(0, w)], out_ref.at[:, col],
                out_sems.at[slot],
            )
            cp.start()
            out_cps[c] = cp
            ag = pltpu.make_async_remote_copy(
                src_ref=red_stage.at[slot, :, pl.ds(0, w)],
                dst_ref=out_ref.at[:, col],
                send_sem=ag_send_sems.at[c],
                recv_sem=ag_recv_sems.at[c],
                device_id=x_nbr,
                device_id_type=_DeviceIdType.MESH,
            )
            ag.start()
            ag_ops[c] = ag

        dy_cps[0].wait()
        b = dy_vmem[0, :, :].astype(jnp.bfloat16)
        dy_cps[2] = dy_fetch(2)
        mm_half(0, b, (1 - my) * D_HALF)
        _sem_wait(barrier, 2)
        rs_start(0)
        x_cp3 = x_fetch(3)
        x_cp2.wait()
        x_cast(2)
        x_cp3.wait()
        x_cast(3)
        mm_half(0, b, my * D_HALF)

        for c in range(1, NCH):
            dy_cps[c].wait()
            w = CHUNKS[c][1]
            b = dy_vmem[c % 3, :, pl.ds(0, w)].astype(jnp.bfloat16)
            if c + 2 < NCH:
                dy_cps[c + 2] = dy_fetch(c + 2)
            if c >= NPB:
                rs_ops[c - NPB].wait_send()
            mm_half(c, b, (1 - my) * D_HALF)
            rs_start(c)
            mm_half(c, b, my * D_HALF)
            if c >= 2:
                finish(c - 2)
        finish(NCH - 2)
        finish(NCH - 1)

        for c in range(NCH - NPB, NCH):
            rs_ops[c].wait_send()
            out_cps[c].wait()
            ag_ops[c].wait_send()
        for c in range(NCH):
            ag_ops[c].wait_recv()

    return pl.pallas_call(
        body,
        out_shape=jax.ShapeDtypeStruct((D_HALF, F), jnp.bfloat16),
        in_specs=[
            pl.BlockSpec(memory_space=pl.ANY),
            pl.BlockSpec(memory_space=pl.ANY),
        ],
        out_specs=pl.BlockSpec(memory_space=pl.ANY),
        scratch_shapes=[
            pltpu.VMEM((M, D), jnp.bfloat16),
            pltpu.VMEM((2, M, XT), jnp.float32),
            pltpu.VMEM((3, M, FC), jnp.float32),
            pltpu.VMEM((NPB, D, FC), jnp.bfloat16),
            pltpu.VMEM((NPB, D_HALF, FC), jnp.bfloat16),
            pltpu.VMEM((NCH, D_HALF, FC), jnp.bfloat16),
            pltpu.SemaphoreType.DMA((2,)),
            pltpu.SemaphoreType.DMA((3,)),
            pltpu.SemaphoreType.DMA((NPB,)),
            pltpu.SemaphoreType.DMA((NCH,)),
            pltpu.SemaphoreType.DMA((NCH,)),
            pltpu.SemaphoreType.DMA((NCH,)),
            pltpu.SemaphoreType.DMA((NCH,)),
        ],
        compiler_params=pltpu.CompilerParams(
            collective_id=0,
            vmem_limit_bytes=60 * 1024 * 1024,
        ),
    )(x, dy)
